# Initial kernel scaffold; baseline (speedup 1.0000x reference)
import jax
import jax.numpy as jnp
from jax.experimental import pallas as pl


def kernel(x, edge_index, batch, W1, b1, W2, b2):
    raise NotImplementedError("write your pallas kernel here")


# speedup vs baseline: 20.4092x; 20.4092x over previous
import functools

import jax
import jax.numpy as jnp
from jax import lax
from jax.experimental import pallas as pl
from jax.experimental.pallas import tpu as pltpu
from jax.experimental.pallas import tpu_sc as plsc

N = 10000
E = 320000
F = 128
G = 64

NC, NS = 2, 16
NW = NC * NS
EPW = E // NW
CHUNK = 80
NCHUNK = EPW // CHUNK
NP = 10240
RPS = NP // NS
ZROWS = 128
DPS = NP // NS


def _deg_sc_body(dst_hbm, out_hbm, zbuf, ones_v, didx, acc):
    c = lax.axis_index("c")
    s = lax.axis_index("s")

    def _zfill(i, carry):
        zbuf[pl.ds(i * 16, 16)] = jnp.zeros((16,), jnp.float32)
        return carry

    lax.fori_loop(0, DPS // 16, _zfill, 0)
    for j in range(CHUNK // 16):
        ones_v[pl.ds(j * 16, 16)] = jnp.ones((16,), jnp.float32)
    pltpu.sync_copy(zbuf, acc.at[pl.ds(s * DPS, DPS)])
    plsc.subcore_barrier()

    base = (c * NS + s) * EPW

    def _body(b, carry):
        pltpu.sync_copy(dst_hbm.at[pl.ds(base + b * CHUNK, CHUNK)], didx.at[0])
        pltpu.sync_copy(ones_v, acc.at[didx.at[0]], add=True)
        return carry

    lax.fori_loop(0, NCHUNK, _body, 0)
    plsc.subcore_barrier()
    pltpu.sync_copy(acc.at[pl.ds(s * DPS, DPS)], zbuf)
    pltpu.sync_copy(zbuf, out_hbm.at[pl.ds(c * NP + s * DPS, DPS)])


def _agg_sc_body(g_hbm, src_hbm, dst_hbm, out_hbm, zbuf, sidx, didx, rows, sem, acc):
    c = lax.axis_index("c")
    s = lax.axis_index("s")

    def _zrow(i, carry):
        for j in range(F // 16):
            zbuf[i, pl.ds(j * 16, 16)] = jnp.zeros((16,), jnp.float32)
        return carry

    lax.fori_loop(0, ZROWS, _zrow, 0)
    for t in range(RPS // ZROWS):
        pltpu.sync_copy(zbuf, acc.at[pl.ds(s * RPS + t * ZROWS, ZROWS)])
    plsc.subcore_barrier()

    base = (c * NS + s) * EPW
    pltpu.sync_copy(src_hbm.at[pl.ds(base, CHUNK)], sidx.at[0])
    pltpu.sync_copy(dst_hbm.at[pl.ds(base, CHUNK)], didx.at[0])
    pltpu.async_copy(g_hbm.at[sidx.at[0]], rows.at[0], sem)

    def _body(b, carry):
        p = lax.rem(b, 2)
        q = lax.rem(b + 1, 2)

        @pl.when(b + 1 < NCHUNK)
        def _prefetch():
            off = base + (b + 1) * CHUNK
            pltpu.sync_copy(src_hbm.at[pl.ds(off, CHUNK)], sidx.at[q])
            pltpu.sync_copy(dst_hbm.at[pl.ds(off, CHUNK)], didx.at[q])
            pltpu.async_copy(g_hbm.at[sidx.at[q]], rows.at[q], sem)

        pltpu.make_async_copy(g_hbm.at[sidx.at[p]], rows.at[p], sem).wait()
        pltpu.sync_copy(rows.at[p], acc.at[didx.at[p]], add=True)
        return carry

    lax.fori_loop(0, NCHUNK, _body, 0)
    plsc.subcore_barrier()

    for t in range(RPS // ZROWS):
        r0 = s * RPS + t * ZROWS
        pltpu.sync_copy(acc.at[pl.ds(r0, ZROWS)], zbuf)
        pltpu.sync_copy(zbuf, out_hbm.at[pl.ds(c * NP + r0, ZROWS)])


@functools.cache
def _sc_kernels():
    mesh = plsc.VectorSubcoreMesh(
        core_axis_name="c", subcore_axis_name="s",
        num_cores=NC, num_subcores=NS)
    deg = pl.kernel(
        _deg_sc_body,
        out_type=jax.ShapeDtypeStruct((NC * NP,), jnp.float32),
        mesh=mesh,
        scratch_types=[
            pltpu.VMEM((DPS,), jnp.float32),
            pltpu.VMEM((CHUNK,), jnp.float32),
            pltpu.VMEM((1, CHUNK), jnp.int32),
            pltpu.VMEM_SHARED((NP,), jnp.float32),
        ],
    )
    agg = pl.kernel(
        _agg_sc_body,
        out_type=jax.ShapeDtypeStruct((NC * NP, F), jnp.float32),
        mesh=mesh,
        scratch_types=[
            pltpu.VMEM((ZROWS, F), jnp.float32),
            pltpu.VMEM((2, CHUNK), jnp.int32),
            pltpu.VMEM((2, CHUNK), jnp.int32),
            pltpu.VMEM((2, CHUNK, F), jnp.float32),
            pltpu.SemaphoreType.DMA,
            pltpu.VMEM_SHARED((NP, F), jnp.float32),
        ],
    )
    return deg, agg


def _tc1_body(pdeg_ref, x_ref, w1_ref, g_ref, dinv_ref):
    deg = pdeg_ref[0:N] + pdeg_ref[NP:NP + N] + 1.0
    dinv = lax.rsqrt(deg)
    h = jnp.dot(x_ref[...], w1_ref[...], preferred_element_type=jnp.float32)
    g_ref[...] = h * dinv
    dinv_ref[...] = dinv


_tc1 = pl.pallas_call(
    _tc1_body,
    out_shape=(
        jax.ShapeDtypeStruct((N, F), jnp.float32),
        jax.ShapeDtypeStruct((N, 1), jnp.float32),
    ),
)


def _tc2_body(pagg_ref, g_ref, dinv_ref, b1_ref, w2_ref, g2_ref):
    agg = pagg_ref[0:N] + pagg_ref[NP:NP + N] + g_ref[...]
    a = jnp.maximum(agg * dinv_ref[...] + b1_ref[...], 0.0)
    h2 = jnp.dot(a, w2_ref[...], preferred_element_type=jnp.float32)
    g2_ref[...] = h2 * dinv_ref[...]


_tc2 = pl.pallas_call(
    _tc2_body,
    out_shape=jax.ShapeDtypeStruct((N, F), jnp.float32),
)


def _tc3_body(pagg_ref, g2_ref, dinv_ref, b2_ref, batch_ref, out_ref):
    agg = pagg_ref[0:N] + pagg_ref[NP:NP + N] + g2_ref[...]
    a = jnp.maximum(agg * dinv_ref[...] + b2_ref[...], 0.0)
    gids = lax.broadcasted_iota(jnp.int32, (G, N), 0)
    mask = (batch_ref[...] == gids).astype(jnp.float32)
    sums = jnp.dot(mask, a, preferred_element_type=jnp.float32)
    counts = jnp.sum(mask, axis=1, keepdims=True)
    out_ref[...] = sums / jnp.maximum(counts, 1.0)


_tc3 = pl.pallas_call(
    _tc3_body,
    out_shape=jax.ShapeDtypeStruct((G, F), jnp.float32),
)


def kernel(x, edge_index, batch, W1, b1, W2, b2):
    _deg_sc, _agg_sc = _sc_kernels()
    src = edge_index[0]
    dst = edge_index[1]
    pdeg = _deg_sc(dst)
    g1, dinv = _tc1(pdeg.reshape(NC * NP, 1), x, W1)
    pagg1 = _agg_sc(g1, src, dst)
    g2 = _tc2(pagg1, g1, dinv, b1.reshape(1, F), W2)
    pagg2 = _agg_sc(g2, src, dst)
    return _tc3(pagg2, g2, dinv, b2.reshape(1, F), batch.reshape(1, N))


# speedup vs baseline: 32.8807x; 1.6111x over previous
import functools

import jax
import jax.numpy as jnp
from jax import lax
from jax.experimental import pallas as pl
from jax.experimental.pallas import tpu as pltpu
from jax.experimental.pallas import tpu_sc as plsc

N = 10000
E = 320000
F = 128
G = 64

NC, NS = 2, 16
NW = NC * NS
EPW = E // NW
CHUNK = 80
NCHUNK = EPW // CHUNK
NP = 10240
RPS = NP // NS
NBUF = 2


DW = 8


def _deg_sc_body(di_hbm, out_hbm, zbuf, ones_v, didx, sem, acc):
    c = lax.axis_index("c")
    s = lax.axis_index("s")
    wid = c * NS + s
    dps = NP // NS

    def _zfill(i, carry):
        zbuf[pl.ds(i * 16, 16)] = jnp.zeros((16,), jnp.float32)
        return carry

    lax.fori_loop(0, dps // 16, _zfill, 0)
    for j in range(CHUNK // 16):
        ones_v[pl.ds(j * 16, 16)] = jnp.ones((16,), jnp.float32)
    pltpu.sync_copy(zbuf, acc.at[pl.ds(s * dps, dps)])
    plsc.subcore_barrier()

    pltpu.sync_copy(di_hbm.at[wid], didx)

    def _wait_one():
        pltpu.make_async_copy(ones_v, acc.at[didx.at[0]], sem).wait()

    def _body(b, carry):
        pltpu.async_copy(ones_v, acc.at[didx.at[b]], sem, add=True)

        @pl.when(b >= DW)
        def _wait():
            _wait_one()

        return carry

    lax.fori_loop(0, NCHUNK, _body, 0)
    for _ in range(DW):
        _wait_one()
    plsc.subcore_barrier()
    pltpu.sync_copy(acc.at[pl.ds(s * dps, dps)],
                    out_hbm.at[pl.ds(c * NP + s * dps, dps)])


def _agg_sc_body(g_hbm, si_hbm, di_hbm, out_hbm, sidx, didx, rows,
                 sem_g, sem_s, acc):
    c = lax.axis_index("c")
    s = lax.axis_index("s")
    wid = c * NS + s

    def _zrow(i, carry):
        for j in range(F // 16):
            rows[0, i, pl.ds(j * 16, 16)] = jnp.zeros((16,), jnp.float32)
        return carry

    lax.fori_loop(0, CHUNK, _zrow, 0)
    for t in range(RPS // CHUNK):
        pltpu.sync_copy(rows.at[0], acc.at[pl.ds(s * RPS + t * CHUNK, CHUNK)])
    plsc.subcore_barrier()

    pltpu.sync_copy(si_hbm.at[pl.ds(wid * EPW, EPW)], sidx)
    pltpu.sync_copy(di_hbm.at[wid], didx)

    def _scat_wait():
        pltpu.make_async_copy(rows.at[0], acc.at[didx.at[0]], sem_s).wait()

    def _gather(b, p):
        pltpu.async_copy(
            g_hbm.at[sidx.at[pl.ds(b * CHUNK, CHUNK)]], rows.at[p], sem_g)

    for j in range(NBUF - 1):
        _gather(j, j)

    def _body(b, carry):
        @pl.when(b >= 1)
        def _wait_prev_scatter():
            _scat_wait()

        @pl.when(b + NBUF - 1 < NCHUNK)
        def _prefetch():
            _gather(b + NBUF - 1, lax.rem(b + NBUF - 1, NBUF))

        p = lax.rem(b, NBUF)
        pltpu.make_async_copy(
            g_hbm.at[sidx.at[pl.ds(b * CHUNK, CHUNK)]], rows.at[p], sem_g).wait()
        pltpu.async_copy(rows.at[p], acc.at[didx.at[b]], sem_s, add=True)
        return carry

    lax.fori_loop(0, NCHUNK, _body, 0)
    _scat_wait()
    plsc.subcore_barrier()

    for t in range(RPS // CHUNK):
        r0 = s * RPS + t * CHUNK
        pltpu.sync_copy(acc.at[pl.ds(r0, CHUNK)], out_hbm.at[pl.ds(c * NP + r0, CHUNK)])


@functools.cache
def _sc_kernels():
    mesh = plsc.VectorSubcoreMesh(
        core_axis_name="c", subcore_axis_name="s",
        num_cores=NC, num_subcores=NS)
    deg = pl.kernel(
        _deg_sc_body,
        out_type=jax.ShapeDtypeStruct((NC * NP,), jnp.float32),
        mesh=mesh,
        scratch_types=[
            pltpu.VMEM((NP // NS,), jnp.float32),
            pltpu.VMEM((CHUNK,), jnp.float32),
            pltpu.VMEM((NCHUNK, CHUNK), jnp.int32),
            pltpu.SemaphoreType.DMA,
            pltpu.VMEM_SHARED((NP,), jnp.float32),
        ],
    )
    agg = pl.kernel(
        _agg_sc_body,
        out_type=jax.ShapeDtypeStruct((NC * NP, F), jnp.float32),
        mesh=mesh,
        scratch_types=[
            pltpu.VMEM((EPW,), jnp.int32),
            pltpu.VMEM((NCHUNK, CHUNK), jnp.int32),
            pltpu.VMEM((NBUF, CHUNK, F), jnp.float32),
            pltpu.SemaphoreType.DMA,
            pltpu.SemaphoreType.DMA,
            pltpu.VMEM_SHARED((NP, F), jnp.float32),
        ],
    )
    return deg, agg


def _tc1_body(pdeg_ref, x_ref, w1_ref, g_ref, dinv_ref):
    deg = pdeg_ref[0:N] + pdeg_ref[NP:NP + N] + 1.0
    dinv = lax.rsqrt(deg)
    h = jnp.dot(x_ref[...], w1_ref[...], preferred_element_type=jnp.float32)
    g_ref[...] = h * dinv
    dinv_ref[...] = dinv


_tc1 = pl.pallas_call(
    _tc1_body,
    out_shape=(
        jax.ShapeDtypeStruct((N, F), jnp.float32),
        jax.ShapeDtypeStruct((N, 1), jnp.float32),
    ),
)


def _tc2_body(pagg_ref, g_ref, dinv_ref, b1_ref, w2_ref, g2_ref):
    agg = pagg_ref[0:N] + pagg_ref[NP:NP + N] + g_ref[...]
    a = jnp.maximum(agg * dinv_ref[...] + b1_ref[...], 0.0)
    h2 = jnp.dot(a, w2_ref[...], preferred_element_type=jnp.float32)
    g2_ref[...] = h2 * dinv_ref[...]


_tc2 = pl.pallas_call(
    _tc2_body,
    out_shape=jax.ShapeDtypeStruct((N, F), jnp.float32),
)


def _tc3_body(pagg_ref, g2_ref, dinv_ref, b2_ref, batch_ref, out_ref):
    agg = pagg_ref[0:N] + pagg_ref[NP:NP + N] + g2_ref[...]
    a = jnp.maximum(agg * dinv_ref[...] + b2_ref[...], 0.0)
    gids = lax.broadcasted_iota(jnp.int32, (G, N), 0)
    mask = (batch_ref[...] == gids).astype(jnp.float32)
    sums = jnp.dot(mask, a, preferred_element_type=jnp.float32)
    counts = jnp.sum(mask, axis=1, keepdims=True)
    out_ref[...] = sums / jnp.maximum(counts, 1.0)


_tc3 = pl.pallas_call(
    _tc3_body,
    out_shape=jax.ShapeDtypeStruct((G, F), jnp.float32),
)


def kernel(x, edge_index, batch, W1, b1, W2, b2):
    _deg_sc, _agg_sc = _sc_kernels()
    src = edge_index[0]
    dst = edge_index[1]
    dst3 = dst.reshape(NW, NCHUNK, CHUNK)
    pdeg = _deg_sc(dst3)
    g1, dinv = _tc1(pdeg.reshape(NC * NP, 1), x, W1)
    pagg1 = _agg_sc(g1, src, dst3)
    g2 = _tc2(pagg1, g1, dinv, b1.reshape(1, F), W2)
    pagg2 = _agg_sc(g2, src, dst3)
    return _tc3(pagg2, g2, dinv, b2.reshape(1, F), batch.reshape(1, N))


# speedup vs baseline: 33.3838x; 1.0153x over previous
import functools

import jax
import jax.numpy as jnp
from jax import lax
from jax.experimental import pallas as pl
from jax.experimental.pallas import tpu as pltpu
from jax.experimental.pallas import tpu_sc as plsc

N = 10000
E = 320000
F = 128
G = 64

NC, NS = 2, 16
NW = NC * NS
EPW = E // NW
CHUNK = 80
NCHUNK = EPW // CHUNK
NP = 10240
RPS = NP // NS
NBUF = 2


DW = 8


def _deg_sc_body(di_hbm, out_hbm, zbuf, ones_v, didx, sem, acc):
    c = lax.axis_index("c")
    s = lax.axis_index("s")
    wid = c * NS + s
    dps = NP // NS

    def _zfill(i, carry):
        zbuf[pl.ds(i * 16, 16)] = jnp.zeros((16,), jnp.float32)
        return carry

    lax.fori_loop(0, dps // 16, _zfill, 0)
    for j in range(CHUNK // 16):
        ones_v[pl.ds(j * 16, 16)] = jnp.ones((16,), jnp.float32)
    pltpu.sync_copy(zbuf, acc.at[pl.ds(s * dps, dps)])
    plsc.subcore_barrier()

    pltpu.sync_copy(di_hbm.at[wid], didx)

    def _wait_one():
        pltpu.make_async_copy(ones_v, acc.at[didx.at[0]], sem).wait()

    def _body(b, carry):
        pltpu.async_copy(ones_v, acc.at[didx.at[b]], sem, add=True)

        @pl.when(b >= DW)
        def _wait():
            _wait_one()

        return carry

    lax.fori_loop(0, NCHUNK, _body, 0)
    for _ in range(DW):
        _wait_one()
    plsc.subcore_barrier()
    pltpu.sync_copy(acc.at[pl.ds(s * dps, dps)],
                    out_hbm.at[pl.ds(c * NP + s * dps, dps)])


def _agg_sc_body(g_hbm, si_hbm, di_hbm, out_hbm, sidx, didx, rows,
                 sem_g, sem_s, acc):
    c = lax.axis_index("c")
    s = lax.axis_index("s")
    wid = c * NS + s

    cp_si = pltpu.async_copy(si_hbm.at[pl.ds(wid * EPW, EPW)], sidx, sem_g)
    cp_di = pltpu.async_copy(di_hbm.at[wid], didx, sem_g)

    def _zrow(i, carry):
        for j in range(F // 16):
            rows[0, i, pl.ds(j * 16, 16)] = jnp.zeros((16,), jnp.float32)
        return carry

    lax.fori_loop(0, CHUNK, _zrow, 0)
    zcps = [pltpu.async_copy(rows.at[0],
                             acc.at[pl.ds(s * RPS + t * CHUNK, CHUNK)], sem_s)
            for t in range(RPS // CHUNK)]
    for cp in zcps:
        cp.wait()
    cp_si.wait()
    cp_di.wait()
    plsc.subcore_barrier()

    def _scat_wait():
        pltpu.make_async_copy(rows.at[0], acc.at[didx.at[0]], sem_s).wait()

    def _gather(b, p):
        pltpu.async_copy(
            g_hbm.at[sidx.at[pl.ds(b * CHUNK, CHUNK)]], rows.at[p], sem_g)

    for j in range(NBUF - 1):
        _gather(j, j)

    def _body(b, carry):
        @pl.when(b >= 1)
        def _wait_prev_scatter():
            _scat_wait()

        @pl.when(b + NBUF - 1 < NCHUNK)
        def _prefetch():
            _gather(b + NBUF - 1, lax.rem(b + NBUF - 1, NBUF))

        p = lax.rem(b, NBUF)
        pltpu.make_async_copy(
            g_hbm.at[sidx.at[pl.ds(b * CHUNK, CHUNK)]], rows.at[p], sem_g).wait()
        pltpu.async_copy(rows.at[p], acc.at[didx.at[b]], sem_s, add=True)
        return carry

    lax.fori_loop(0, NCHUNK, _body, 0)
    _scat_wait()
    plsc.subcore_barrier()

    ocps = [pltpu.async_copy(acc.at[pl.ds(s * RPS + t * CHUNK, CHUNK)],
                             out_hbm.at[pl.ds(c * NP + s * RPS + t * CHUNK, CHUNK)],
                             sem_s)
            for t in range(RPS // CHUNK)]
    for cp in ocps:
        cp.wait()


@functools.cache
def _sc_kernels():
    mesh = plsc.VectorSubcoreMesh(
        core_axis_name="c", subcore_axis_name="s",
        num_cores=NC, num_subcores=NS)
    deg = pl.kernel(
        _deg_sc_body,
        out_type=jax.ShapeDtypeStruct((NC * NP,), jnp.float32),
        mesh=mesh,
        scratch_types=[
            pltpu.VMEM((NP // NS,), jnp.float32),
            pltpu.VMEM((CHUNK,), jnp.float32),
            pltpu.VMEM((NCHUNK, CHUNK), jnp.int32),
            pltpu.SemaphoreType.DMA,
            pltpu.VMEM_SHARED((NP,), jnp.float32),
        ],
    )
    agg = pl.kernel(
        _agg_sc_body,
        out_type=jax.ShapeDtypeStruct((NC * NP, F), jnp.float32),
        mesh=mesh,
        scratch_types=[
            pltpu.VMEM((EPW,), jnp.int32),
            pltpu.VMEM((NCHUNK, CHUNK), jnp.int32),
            pltpu.VMEM((NBUF, CHUNK, F), jnp.float32),
            pltpu.SemaphoreType.DMA,
            pltpu.SemaphoreType.DMA,
            pltpu.VMEM_SHARED((NP, F), jnp.float32),
        ],
    )
    return deg, agg


def _tc1_body(pdeg_ref, x_ref, w1_ref, g_ref, dinv_ref):
    deg = pdeg_ref[0:N] + pdeg_ref[NP:NP + N] + 1.0
    dinv = lax.rsqrt(deg)
    h = jnp.dot(x_ref[...], w1_ref[...], preferred_element_type=jnp.float32)
    g_ref[...] = h * dinv
    dinv_ref[...] = dinv


_tc1 = pl.pallas_call(
    _tc1_body,
    out_shape=(
        jax.ShapeDtypeStruct((N, F), jnp.float32),
        jax.ShapeDtypeStruct((N, 1), jnp.float32),
    ),
)


def _tc2_body(pagg_ref, g_ref, dinv_ref, b1_ref, w2_ref, g2_ref):
    agg = pagg_ref[0:N] + pagg_ref[NP:NP + N] + g_ref[...]
    a = jnp.maximum(agg * dinv_ref[...] + b1_ref[...], 0.0)
    h2 = jnp.dot(a, w2_ref[...], preferred_element_type=jnp.float32)
    g2_ref[...] = h2 * dinv_ref[...]


_tc2 = pl.pallas_call(
    _tc2_body,
    out_shape=jax.ShapeDtypeStruct((N, F), jnp.float32),
)


def _tc3_body(pagg_ref, g2_ref, dinv_ref, b2_ref, batch_ref, out_ref):
    agg = pagg_ref[0:N] + pagg_ref[NP:NP + N] + g2_ref[...]
    a = jnp.maximum(agg * dinv_ref[...] + b2_ref[...], 0.0)
    gids = lax.broadcasted_iota(jnp.int32, (G, N), 0)
    mask = (batch_ref[...] == gids).astype(jnp.float32)
    sums = jnp.dot(mask, a, preferred_element_type=jnp.float32)
    counts = jnp.sum(mask, axis=1, keepdims=True)
    out_ref[...] = sums / jnp.maximum(counts, 1.0)


_tc3 = pl.pallas_call(
    _tc3_body,
    out_shape=jax.ShapeDtypeStruct((G, F), jnp.float32),
)


def kernel(x, edge_index, batch, W1, b1, W2, b2):
    _deg_sc, _agg_sc = _sc_kernels()
    src = edge_index[0]
    dst = edge_index[1]
    dst3 = dst.reshape(NW, NCHUNK, CHUNK)
    pdeg = _deg_sc(dst3)
    g1, dinv = _tc1(pdeg.reshape(NC * NP, 1), x, W1)
    pagg1 = _agg_sc(g1, src, dst3)
    g2 = _tc2(pagg1, g1, dinv, b1.reshape(1, F), W2)
    pagg2 = _agg_sc(g2, src, dst3)
    return _tc3(pagg2, g2, dinv, b2.reshape(1, F), batch.reshape(1, N))


# speedup vs baseline: 39.6837x; 1.1887x over previous
import functools

import jax
import jax.numpy as jnp
from jax import lax
from jax.experimental import pallas as pl
from jax.experimental.pallas import tpu as pltpu
from jax.experimental.pallas import tpu_sc as plsc

N = 10000
E = 320000
F = 128
G = 64

NC, NS = 2, 16
NW = NC * NS
EPW = E // NW
CHUNK = 40
NCHUNK = EPW // CHUNK
NP = 10240
RPS = NP // NS
NBUF = 5


DW = 8


def _deg_sc_body(di_hbm, out_hbm, zbuf, ones_v, didx, sem, acc):
    c = lax.axis_index("c")
    s = lax.axis_index("s")
    wid = c * NS + s
    dps = NP // NS

    def _zfill(i, carry):
        zbuf[pl.ds(i * 16, 16)] = jnp.zeros((16,), jnp.float32)
        return carry

    lax.fori_loop(0, dps // 16, _zfill, 0)
    for j in range(CHUNK // 16):
        ones_v[pl.ds(j * 16, 16)] = jnp.ones((16,), jnp.float32)
    pltpu.sync_copy(zbuf, acc.at[pl.ds(s * dps, dps)])
    plsc.subcore_barrier()

    pltpu.sync_copy(di_hbm.at[wid], didx)

    def _wait_one():
        pltpu.make_async_copy(ones_v, acc.at[didx.at[0]], sem).wait()

    def _body(b, carry):
        pltpu.async_copy(ones_v, acc.at[didx.at[b]], sem, add=True)

        @pl.when(b >= DW)
        def _wait():
            _wait_one()

        return carry

    lax.fori_loop(0, NCHUNK, _body, 0)
    for _ in range(DW):
        _wait_one()
    plsc.subcore_barrier()
    pltpu.sync_copy(acc.at[pl.ds(s * dps, dps)],
                    out_hbm.at[pl.ds(c * NP + s * dps, dps)])


def _agg_sc_body(g_hbm, si_hbm, di_hbm, out_hbm, sidx, didx, rows,
                 sem_g, sem_s, acc):
    c = lax.axis_index("c")
    s = lax.axis_index("s")
    wid = c * NS + s

    cp_si = pltpu.async_copy(si_hbm.at[pl.ds(wid * EPW, EPW)], sidx, sem_g)
    cp_di = pltpu.async_copy(di_hbm.at[pl.ds(wid * EPW, EPW)], didx, sem_g)

    def _zrow(i, carry):
        for j in range(F // 16):
            rows[0, i, pl.ds(j * 16, 16)] = jnp.zeros((16,), jnp.float32)
        return carry

    lax.fori_loop(0, CHUNK, _zrow, 0)
    zcps = [pltpu.async_copy(rows.at[0],
                             acc.at[pl.ds(s * RPS + t * CHUNK, CHUNK)], sem_s)
            for t in range(RPS // CHUNK)]
    for cp in zcps:
        cp.wait()
    cp_si.wait()
    cp_di.wait()
    plsc.subcore_barrier()

    def _scat_wait():
        pltpu.make_async_copy(
            rows.at[0], acc.at[didx.at[pl.ds(0, CHUNK)]], sem_s).wait()

    def _gather(b, p):
        pltpu.async_copy(
            g_hbm.at[sidx.at[pl.ds(b * CHUNK, CHUNK)]], rows.at[p], sem_g)

    for j in range(NBUF - 1):
        _gather(j, j)

    def _body(b, carry):
        @pl.when(b >= 1)
        def _wait_prev_scatter():
            _scat_wait()

        @pl.when(b + NBUF - 1 < NCHUNK)
        def _prefetch():
            _gather(b + NBUF - 1, lax.rem(b + NBUF - 1, NBUF))

        p = lax.rem(b, NBUF)
        pltpu.make_async_copy(
            g_hbm.at[sidx.at[pl.ds(b * CHUNK, CHUNK)]], rows.at[p], sem_g).wait()
        pltpu.async_copy(
            rows.at[p], acc.at[didx.at[pl.ds(b * CHUNK, CHUNK)]], sem_s,
            add=True)
        return carry

    lax.fori_loop(0, NCHUNK, _body, 0)
    _scat_wait()
    plsc.subcore_barrier()

    ocps = [pltpu.async_copy(acc.at[pl.ds(s * RPS + t * CHUNK, CHUNK)],
                             out_hbm.at[pl.ds(c * NP + s * RPS + t * CHUNK, CHUNK)],
                             sem_s)
            for t in range(RPS // CHUNK)]
    for cp in ocps:
        cp.wait()


@functools.cache
def _sc_kernels():
    mesh = plsc.VectorSubcoreMesh(
        core_axis_name="c", subcore_axis_name="s",
        num_cores=NC, num_subcores=NS)
    deg = pl.kernel(
        _deg_sc_body,
        out_type=jax.ShapeDtypeStruct((NC * NP,), jnp.float32),
        mesh=mesh,
        scratch_types=[
            pltpu.VMEM((NP // NS,), jnp.float32),
            pltpu.VMEM((CHUNK,), jnp.float32),
            pltpu.VMEM((NCHUNK, CHUNK), jnp.int32),
            pltpu.SemaphoreType.DMA,
            pltpu.VMEM_SHARED((NP,), jnp.float32),
        ],
    )
    agg = pl.kernel(
        _agg_sc_body,
        out_type=jax.ShapeDtypeStruct((NC * NP, F), jnp.float32),
        mesh=mesh,
        scratch_types=[
            pltpu.VMEM((EPW,), jnp.int32),
            pltpu.VMEM((EPW,), jnp.int32),
            pltpu.VMEM((NBUF, CHUNK, F), jnp.float32),
            pltpu.SemaphoreType.DMA,
            pltpu.SemaphoreType.DMA,
            pltpu.VMEM_SHARED((NP, F), jnp.float32),
        ],
    )
    return deg, agg


def _tc1_body(pdeg_ref, x_ref, w1_ref, g_ref, dinv_ref):
    deg = pdeg_ref[0:N] + pdeg_ref[NP:NP + N] + 1.0
    dinv = lax.rsqrt(deg)
    h = jnp.dot(x_ref[...], w1_ref[...], preferred_element_type=jnp.float32)
    g_ref[...] = h * dinv
    dinv_ref[...] = dinv


_tc1 = pl.pallas_call(
    _tc1_body,
    out_shape=(
        jax.ShapeDtypeStruct((N, F), jnp.float32),
        jax.ShapeDtypeStruct((N, 1), jnp.float32),
    ),
)


def _tc2_body(pagg_ref, g_ref, dinv_ref, b1_ref, w2_ref, g2_ref):
    agg = pagg_ref[0:N] + pagg_ref[NP:NP + N] + g_ref[...]
    a = jnp.maximum(agg * dinv_ref[...] + b1_ref[...], 0.0)
    h2 = jnp.dot(a, w2_ref[...], preferred_element_type=jnp.float32)
    g2_ref[...] = h2 * dinv_ref[...]


_tc2 = pl.pallas_call(
    _tc2_body,
    out_shape=jax.ShapeDtypeStruct((N, F), jnp.float32),
)


def _tc3_body(pagg_ref, g2_ref, dinv_ref, b2_ref, batch_ref, out_ref):
    agg = pagg_ref[0:N] + pagg_ref[NP:NP + N] + g2_ref[...]
    a = jnp.maximum(agg * dinv_ref[...] + b2_ref[...], 0.0)
    gids = lax.broadcasted_iota(jnp.int32, (G, N), 0)
    mask = (batch_ref[...] == gids).astype(jnp.float32)
    sums = jnp.dot(mask, a, preferred_element_type=jnp.float32)
    counts = jnp.sum(mask, axis=1, keepdims=True)
    out_ref[...] = sums / jnp.maximum(counts, 1.0)


_tc3 = pl.pallas_call(
    _tc3_body,
    out_shape=jax.ShapeDtypeStruct((G, F), jnp.float32),
)


def kernel(x, edge_index, batch, W1, b1, W2, b2):
    _deg_sc, _agg_sc = _sc_kernels()
    src = edge_index[0]
    dst = edge_index[1]
    dst3 = dst.reshape(NW, NCHUNK, CHUNK)
    pdeg = _deg_sc(dst3)
    g1, dinv = _tc1(pdeg.reshape(NC * NP, 1), x, W1)
    pagg1 = _agg_sc(g1, src, dst)
    g2 = _tc2(pagg1, g1, dinv, b1.reshape(1, F), W2)
    pagg2 = _agg_sc(g2, src, dst)
    return _tc3(pagg2, g2, dinv, b2.reshape(1, F), batch.reshape(1, N))
